# baseline (speedup 1.0000x reference)
import jax
import jax.numpy as jnp
from jax.experimental import pallas as pl
from jax.experimental.pallas import tpu as pltpu

T = 8192
DM = 1024
E = 64
NA = 32
TILE = 512
NT = T // TILE
_BIG = 2**30


def _router_body(obs_ref, wg_ref, bg_ref, eidx_ref, bv_ref, bi_ref):
    i = pl.program_id(0)
    logits = jnp.dot(obs_ref[...], wg_ref[...],
                     preferred_element_type=jnp.float32) + bg_ref[...]
    m = jnp.max(logits)
    rows = jax.lax.broadcasted_iota(jnp.int32, (TILE, E), 0)
    cols = jax.lax.broadcasted_iota(jnp.int32, (TILE, E), 1)
    flat = (i * TILE + rows) * E + cols
    idx = jnp.min(jnp.where(logits == m, flat, _BIG))

    @pl.when(i == 0)
    def _init():
        bv_ref[0] = m
        bi_ref[0] = idx

    @pl.when(i > 0)
    def _acc():
        bv = bv_ref[0]
        bi = bi_ref[0]
        better = (m > bv) | ((m == bv) & (idx < bi))
        bv_ref[0] = jnp.where(better, m, bv)
        bi_ref[0] = jnp.where(better, idx, bi)

    @pl.when(i == NT - 1)
    def _fin():
        eidx_ref[0] = bi_ref[0] % E


def _expert_body(eidx_ref, obs_ref, we_ref, be_ref, out_ref):
    del eidx_ref
    out_ref[...] = jnp.dot(obs_ref[...], we_ref[0],
                           preferred_element_type=jnp.float32) + be_ref[0]


def kernel(context, obs, Wg, bg, We, be):
    del context
    bg2 = bg.reshape(1, E)
    be3 = be.reshape(E, 1, NA)

    eidx = pl.pallas_call(
        _router_body,
        grid=(NT,),
        in_specs=[
            pl.BlockSpec((TILE, DM), lambda i: (i, 0)),
            pl.BlockSpec((DM, E), lambda i: (0, 0)),
            pl.BlockSpec((1, E), lambda i: (0, 0)),
        ],
        out_specs=pl.BlockSpec(memory_space=pltpu.SMEM),
        out_shape=jax.ShapeDtypeStruct((1,), jnp.int32),
        scratch_shapes=[
            pltpu.SMEM((1,), jnp.float32),
            pltpu.SMEM((1,), jnp.int32),
        ],
        compiler_params=pltpu.CompilerParams(
            dimension_semantics=("arbitrary",),
        ),
    )(obs, Wg, bg2)

    grid_spec = pltpu.PrefetchScalarGridSpec(
        num_scalar_prefetch=1,
        grid=(NT,),
        in_specs=[
            pl.BlockSpec((TILE, DM), lambda i, e: (i, 0)),
            pl.BlockSpec((1, DM, NA), lambda i, e: (e[0], 0, 0)),
            pl.BlockSpec((1, 1, NA), lambda i, e: (e[0], 0, 0)),
        ],
        out_specs=pl.BlockSpec((TILE, NA), lambda i, e: (i, 0)),
    )
    out = pl.pallas_call(
        _expert_body,
        grid_spec=grid_spec,
        out_shape=jax.ShapeDtypeStruct((T, NA), jnp.float32),
        compiler_params=pltpu.CompilerParams(
            dimension_semantics=("arbitrary",),
        ),
    )(eidx, obs, We, be3)
    return out


# speedup vs baseline: 1.1140x; 1.1140x over previous
import jax
import jax.numpy as jnp
from jax.experimental import pallas as pl
from jax.experimental.pallas import tpu as pltpu

T = 8192
DM = 1024
E = 64
NA = 32
TILE = 512
NT = T // TILE
_BIG = 2**30


def _body(obs_ref, wg_ref, bg_ref, we_hbm, be_ref, out_ref,
          obs_save, w_buf, bv_ref, bi_ref, eidx_ref, sem):
    p = pl.program_id(0)
    i = pl.program_id(1)

    @pl.when(p == 0)
    def _phase0():
        x = obs_ref[...]
        obs_save[pl.ds(i * TILE, TILE), :] = x
        logits = jnp.dot(x, wg_ref[...],
                         preferred_element_type=jnp.float32) + bg_ref[...]
        m = jnp.max(logits)
        rows = jax.lax.broadcasted_iota(jnp.int32, (TILE, E), 0)
        cols = jax.lax.broadcasted_iota(jnp.int32, (TILE, E), 1)
        flat = (i * TILE + rows) * E + cols
        idx = jnp.min(jnp.where(logits == m, flat, _BIG))

        @pl.when(i == 0)
        def _init():
            bv_ref[0] = m
            bi_ref[0] = idx

        @pl.when(i > 0)
        def _acc():
            bv = bv_ref[0]
            bi = bi_ref[0]
            better = (m > bv) | ((m == bv) & (idx < bi))
            bv_ref[0] = jnp.where(better, m, bv)
            bi_ref[0] = jnp.where(better, idx, bi)

        @pl.when(i == NT - 1)
        def _route():
            e = bi_ref[0] % E
            eidx_ref[0] = e
            pltpu.make_async_copy(we_hbm.at[e], w_buf, sem).start()

    @pl.when(p == 1)
    def _phase1():
        e = eidx_ref[0]

        @pl.when(i == 0)
        def _wait_w():
            pltpu.make_async_copy(we_hbm.at[e], w_buf, sem).wait()

        x = obs_save[pl.ds(i * TILE, TILE), :]
        rows = jax.lax.broadcasted_iota(jnp.int32, (E, NA), 0)
        b = jnp.sum(jnp.where(rows == e, be_ref[...], 0.0),
                    axis=0, keepdims=True)
        out_ref[...] = jnp.dot(x, w_buf[...],
                               preferred_element_type=jnp.float32) + b


def kernel(context, obs, Wg, bg, We, be):
    del context
    bg2 = bg.reshape(1, E)

    out = pl.pallas_call(
        _body,
        grid=(2, NT),
        in_specs=[
            pl.BlockSpec((TILE, DM), lambda p, i: ((1 - p) * i, 0)),
            pl.BlockSpec((DM, E), lambda p, i: (0, 0)),
            pl.BlockSpec((1, E), lambda p, i: (0, 0)),
            pl.BlockSpec(memory_space=pltpu.MemorySpace.HBM),
            pl.BlockSpec((E, NA), lambda p, i: (0, 0)),
        ],
        out_specs=pl.BlockSpec((TILE, NA), lambda p, i: (p * i, 0)),
        out_shape=jax.ShapeDtypeStruct((T, NA), jnp.float32),
        scratch_shapes=[
            pltpu.VMEM((T, DM), jnp.float32),
            pltpu.VMEM((DM, NA), jnp.float32),
            pltpu.SMEM((1,), jnp.float32),
            pltpu.SMEM((1,), jnp.int32),
            pltpu.SMEM((1,), jnp.int32),
            pltpu.SemaphoreType.DMA,
        ],
        compiler_params=pltpu.CompilerParams(
            dimension_semantics=("arbitrary", "arbitrary"),
        ),
    )(obs, Wg, bg2, We, be)
    return out


# speedup vs baseline: 1.1741x; 1.0540x over previous
import jax
import jax.numpy as jnp
from jax.experimental import pallas as pl
from jax.experimental.pallas import tpu as pltpu

T = 8192
DM = 1024
E = 64
NA = 32
TILE = 1024
NT = T // TILE
_BIG = 2**30


def _body(obs_ref, wg_ref, bg_ref, we_hbm, be_ref, out_ref,
          w_buf, b_buf, bv_ref, bi_ref, eidx_ref, sem):
    p = pl.program_id(0)
    i = pl.program_id(1)

    @pl.when(p == 0)
    def _phase0():
        logits = jnp.dot(obs_ref[...], wg_ref[...],
                         preferred_element_type=jnp.float32) + bg_ref[...]
        m = jnp.max(logits)
        rows = jax.lax.broadcasted_iota(jnp.int32, (TILE, E), 0)
        cols = jax.lax.broadcasted_iota(jnp.int32, (TILE, E), 1)
        flat = (i * TILE + rows) * E + cols
        idx = jnp.min(jnp.where(logits == m, flat, _BIG))

        @pl.when(i == 0)
        def _init():
            bv_ref[0] = m
            bi_ref[0] = idx

        @pl.when(i > 0)
        def _acc():
            bv = bv_ref[0]
            bi = bi_ref[0]
            better = (m > bv) | ((m == bv) & (idx < bi))
            bv_ref[0] = jnp.where(better, m, bv)
            bi_ref[0] = jnp.where(better, idx, bi)

        @pl.when(i == NT - 1)
        def _route():
            e = bi_ref[0] % E
            eidx_ref[0] = e
            pltpu.make_async_copy(we_hbm.at[e], w_buf, sem).start()

    @pl.when(p == 1)
    def _phase1():
        @pl.when(i == 0)
        def _prep():
            e = eidx_ref[0]
            pltpu.make_async_copy(we_hbm.at[e], w_buf, sem).wait()
            rows = jax.lax.broadcasted_iota(jnp.int32, (E, NA), 0)
            b_buf[...] = jnp.sum(jnp.where(rows == e, be_ref[...], 0.0),
                                 axis=0, keepdims=True)

        out_ref[...] = jnp.dot(obs_ref[...], w_buf[...],
                               preferred_element_type=jnp.float32) + b_buf[...]


def kernel(context, obs, Wg, bg, We, be):
    del context
    bg2 = bg.reshape(1, E)

    out = pl.pallas_call(
        _body,
        grid=(2, NT),
        in_specs=[
            pl.BlockSpec((TILE, DM), lambda p, i: (i, 0)),
            pl.BlockSpec((DM, E), lambda p, i: (0, 0)),
            pl.BlockSpec((1, E), lambda p, i: (0, 0)),
            pl.BlockSpec(memory_space=pltpu.MemorySpace.HBM),
            pl.BlockSpec((E, NA), lambda p, i: (0, 0)),
        ],
        out_specs=pl.BlockSpec((TILE, NA), lambda p, i: (p * i, 0)),
        out_shape=jax.ShapeDtypeStruct((T, NA), jnp.float32),
        scratch_shapes=[
            pltpu.VMEM((DM, NA), jnp.float32),
            pltpu.VMEM((1, NA), jnp.float32),
            pltpu.SMEM((1,), jnp.float32),
            pltpu.SMEM((1,), jnp.int32),
            pltpu.SMEM((1,), jnp.int32),
            pltpu.SemaphoreType.DMA,
        ],
        compiler_params=pltpu.CompilerParams(
            dimension_semantics=("arbitrary", "arbitrary"),
        ),
    )(obs, Wg, bg2, We, be)
    return out


# speedup vs baseline: 2.3872x; 2.0333x over previous
import jax
import jax.numpy as jnp
from jax.experimental import pallas as pl
from jax.experimental.pallas import tpu as pltpu

T = 8192
DM = 1024
E = 64
NA = 32
TILE = 1024
NT = T // TILE
_BIG = 2**30

_CONTRACT_MINOR = (((1,), (1,)), ((), ()))


def _body(obs_ref, wgt_ref, bg_ref, wet_hbm, bet_ref, out_ref,
          w_buf, b_buf, bv_ref, bi_ref, eidx_ref, sem):
    p = pl.program_id(0)
    i = pl.program_id(1)

    @pl.when(p == 0)
    def _phase0():
        logits_t = jax.lax.dot_general(
            wgt_ref[...], obs_ref[...], _CONTRACT_MINOR,
            preferred_element_type=jnp.float32) + bg_ref[...].T
        m = jnp.max(logits_t)
        erow = jax.lax.broadcasted_iota(jnp.int32, (E, TILE), 0)
        tcol = jax.lax.broadcasted_iota(jnp.int32, (E, TILE), 1)
        flat = (i * TILE + tcol) * E + erow
        idx = jnp.min(jnp.where(logits_t == m, flat, _BIG))

        @pl.when(i == 0)
        def _init():
            bv_ref[0] = m
            bi_ref[0] = idx

        @pl.when(i > 0)
        def _acc():
            bv = bv_ref[0]
            bi = bi_ref[0]
            better = (m > bv) | ((m == bv) & (idx < bi))
            bv_ref[0] = jnp.where(better, m, bv)
            bi_ref[0] = jnp.where(better, idx, bi)

        @pl.when(i == NT - 1)
        def _route():
            e = bi_ref[0] % E
            eidx_ref[0] = e
            pltpu.make_async_copy(wet_hbm.at[e], w_buf, sem).start()

    @pl.when(p == 1)
    def _phase1():
        @pl.when(i == 0)
        def _prep():
            e = eidx_ref[0]
            pltpu.make_async_copy(wet_hbm.at[e], w_buf, sem).wait()
            cols = jax.lax.broadcasted_iota(jnp.int32, (NA, E), 1)
            b_buf[...] = jnp.sum(jnp.where(cols == e, bet_ref[...], 0.0),
                                 axis=1, keepdims=True)

        out_ref[...] = jax.lax.dot_general(
            w_buf[...], obs_ref[...], _CONTRACT_MINOR,
            preferred_element_type=jnp.float32) + b_buf[...]


def kernel(context, obs, Wg, bg, We, be):
    del context
    wgt = Wg.T
    wet = jnp.swapaxes(We, 1, 2)
    bet = be.T
    bg2 = bg.reshape(1, E)

    out_t = pl.pallas_call(
        _body,
        grid=(2, NT),
        in_specs=[
            pl.BlockSpec((TILE, DM), lambda p, i: (i, 0)),
            pl.BlockSpec((E, DM), lambda p, i: (0, 0)),
            pl.BlockSpec((1, E), lambda p, i: (0, 0)),
            pl.BlockSpec(memory_space=pltpu.MemorySpace.HBM),
            pl.BlockSpec((NA, E), lambda p, i: (0, 0)),
        ],
        out_specs=pl.BlockSpec((NA, TILE), lambda p, i: (0, p * i)),
        out_shape=jax.ShapeDtypeStruct((NA, T), jnp.float32),
        scratch_shapes=[
            pltpu.VMEM((NA, DM), jnp.float32),
            pltpu.VMEM((NA, 1), jnp.float32),
            pltpu.SMEM((1,), jnp.float32),
            pltpu.SMEM((1,), jnp.int32),
            pltpu.SMEM((1,), jnp.int32),
            pltpu.SemaphoreType.DMA,
        ],
        compiler_params=pltpu.CompilerParams(
            dimension_semantics=("arbitrary", "arbitrary"),
        ),
    )(obs, wgt, bg2, wet, bet)
    return out_t.T


# speedup vs baseline: 3.3214x; 1.3913x over previous
import jax
import jax.numpy as jnp
from jax.experimental import pallas as pl
from jax.experimental.pallas import tpu as pltpu

T = 8192
DM = 1024
E = 64
NA = 32
TILE = 1024
NT = T // TILE
_BIG = 2**30

_CONTRACT_MINOR = (((1,), (1,)), ((), ()))


def _body(obs_hbm, wgt_ref, bg_ref, wet_hbm, bet_ref, out_ref,
          obs_v, w_buf, sems, wsem):
    for i in range(NT):
        pltpu.make_async_copy(
            obs_hbm.at[pl.ds(i * TILE, TILE)],
            obs_v.at[pl.ds(i * TILE, TILE)],
            sems.at[i],
        ).start()

    bgt = bg_ref[...].T
    bv = None
    for i in range(NT):
        pltpu.make_async_copy(
            obs_hbm.at[pl.ds(i * TILE, TILE)],
            obs_v.at[pl.ds(i * TILE, TILE)],
            sems.at[i],
        ).wait()
        x = obs_v[pl.ds(i * TILE, TILE), :]
        logits_t = jax.lax.dot_general(
            wgt_ref[...], x, _CONTRACT_MINOR,
            preferred_element_type=jnp.float32) + bgt
        m = jnp.max(logits_t)
        erow = jax.lax.broadcasted_iota(jnp.int32, (E, TILE), 0)
        tcol = jax.lax.broadcasted_iota(jnp.int32, (E, TILE), 1)
        flat = (i * TILE + tcol) * E + erow
        idx = jnp.min(jnp.where(logits_t == m, flat, _BIG))
        if bv is None:
            bv, bi = m, idx
        else:
            better = (m > bv) | ((m == bv) & (idx < bi))
            bv = jnp.where(better, m, bv)
            bi = jnp.where(better, idx, bi)

    e = bi % E
    pltpu.make_async_copy(wet_hbm.at[e], w_buf, wsem).start()
    cols = jax.lax.broadcasted_iota(jnp.int32, (NA, E), 1)
    b = jnp.sum(jnp.where(cols == e, bet_ref[...], 0.0),
                axis=1, keepdims=True)
    pltpu.make_async_copy(wet_hbm.at[e], w_buf, wsem).wait()

    for i in range(NT):
        x = obs_v[pl.ds(i * TILE, TILE), :]
        out_ref[:, pl.ds(i * TILE, TILE)] = jax.lax.dot_general(
            w_buf[...], x, _CONTRACT_MINOR,
            preferred_element_type=jnp.float32) + b


def kernel(context, obs, Wg, bg, We, be):
    del context
    wgt = Wg.T
    wet = jnp.swapaxes(We, 1, 2)
    bet = be.T
    bg2 = bg.reshape(1, E)

    out_t = pl.pallas_call(
        _body,
        in_specs=[
            pl.BlockSpec(memory_space=pltpu.MemorySpace.HBM),
            pl.BlockSpec((E, DM), lambda: (0, 0)),
            pl.BlockSpec((1, E), lambda: (0, 0)),
            pl.BlockSpec(memory_space=pltpu.MemorySpace.HBM),
            pl.BlockSpec((NA, E), lambda: (0, 0)),
        ],
        out_specs=pl.BlockSpec((NA, T), lambda: (0, 0)),
        out_shape=jax.ShapeDtypeStruct((NA, T), jnp.float32),
        scratch_shapes=[
            pltpu.VMEM((T, DM), jnp.float32),
            pltpu.VMEM((NA, DM), jnp.float32),
            pltpu.SemaphoreType.DMA((NT,)),
            pltpu.SemaphoreType.DMA,
        ],
    )(obs, wgt, bg2, wet, bet)
    return out_t.T
